# baseline (speedup 1.0000x reference)
import jax
import jax.numpy as jnp
from jax.experimental import pallas as pl

_B_TILE = 8192


def _body(a_ref, we1, be1, we2, be2, we3, be3, wd1, bd1, wd2, bd2, wd3, bd3,
          cb_ref, out_ref, idx_ref, rsum_ref, qsum_ref):
    a = a_ref[...]
    h = jax.nn.gelu(jnp.dot(a, we1[...]) + be1[...])
    h = jax.nn.gelu(jnp.dot(h, we2[...]) + be2[...])
    z = jnp.dot(h, we3[...]) + be3[...]
    l_dim = z.shape[1]
    zz = jnp.dot(z * z, jnp.ones((l_dim, 1), jnp.float32))
    rr = 1.0 / (jnp.sqrt(zz) + 1e-8)
    z_e = z * rr
    s = zz * (rr * rr)

    cb = cb_ref[...]
    cb = cb / (jnp.sqrt(jnp.sum(cb * cb, axis=-1, keepdims=True)) + 1e-8)
    n_codes = cb.shape[0]
    half_c = 0.5 * jnp.sum(cb * cb, axis=-1)

    score = jnp.dot(z_e, cb.T) - half_c[None, :]
    m = jnp.max(score, axis=-1, keepdims=True)
    one_hot = (score == m).astype(jnp.float32)

    cbd = jax.nn.gelu(jnp.dot(cb, wd1[...]) + bd1[...])
    cbd = jax.nn.gelu(jnp.dot(cbd, wd2[...]) + bd2[...])
    cb_hat = jnp.dot(cbd, wd3[...]) + bd3[...]
    a_dim = cb_hat.shape[1]
    code_col = jax.lax.broadcasted_iota(jnp.int32, (n_codes, 1), 0).astype(
        jnp.float32)
    pad = jnp.zeros((n_codes, 128 - a_dim - 1), jnp.float32)
    cb_ext = jnp.concatenate([cb_hat, code_col, pad], axis=1)
    gathered = jnp.dot(one_hot, cb_ext)
    a_hat = gathered[:, :a_dim]
    idx = gathered[:, a_dim].astype(jnp.int32)

    out_ref[...] = a_hat
    idx_ref[...] = idx[:, None]

    r_part = jnp.reshape(jnp.sum((a - a_hat) ** 2), (1, 1))
    q_part = jnp.reshape(jnp.sum(s - 2.0 * m), (1, 1))

    @pl.when(pl.program_id(0) == 0)
    def _init():
        rsum_ref[...] = jnp.zeros((1, 1), jnp.float32)
        qsum_ref[...] = jnp.zeros((1, 1), jnp.float32)

    rsum_ref[...] += r_part
    qsum_ref[...] += q_part


def kernel(action, We1, be1, We2, be2, We3, be3, Wd1, bd1, Wd2, bd2, Wd3, bd3,
           codebook):
    n_rows, a_dim = action.shape
    n_codes, l_dim = codebook.shape
    grid = (n_rows // _B_TILE,)

    def _const2(shape):
        return pl.BlockSpec(shape, lambda i: (0, 0))

    biases = [b.reshape(1, -1) for b in (be1, be2, be3, bd1, bd2, bd3)]
    w_list = [We1, We2, We3, Wd1, Wd2, Wd3]
    in_specs = [pl.BlockSpec((_B_TILE, a_dim), lambda i: (i, 0))]
    operands = [action]
    for w, b in zip(w_list, biases):
        in_specs += [_const2(w.shape), _const2(b.shape)]
        operands += [w, b]
    in_specs.append(_const2(codebook.shape))
    operands.append(codebook)

    out_shapes = (
        jax.ShapeDtypeStruct((n_rows, a_dim), jnp.float32),
        jax.ShapeDtypeStruct((n_rows, 1), jnp.int32),
        jax.ShapeDtypeStruct((1, 1), jnp.float32),
        jax.ShapeDtypeStruct((1, 1), jnp.float32),
    )
    out_specs = (
        pl.BlockSpec((_B_TILE, a_dim), lambda i: (i, 0)),
        pl.BlockSpec((_B_TILE, 1), lambda i: (i, 0)),
        _const2((1, 1)),
        _const2((1, 1)),
    )

    a_hat, idx, rsum, qsum = pl.pallas_call(
        _body,
        grid=grid,
        in_specs=in_specs,
        out_specs=out_specs,
        out_shape=out_shapes,
    )(*operands)

    recon_loss = rsum[0, 0] / (n_rows * a_dim)
    q_loss = qsum[0, 0] / (n_rows * l_dim)
    return (a_hat, idx.reshape(n_rows), recon_loss, q_loss, q_loss)


# speedup vs baseline: 1.1356x; 1.1356x over previous
import jax
import jax.numpy as jnp
from jax.experimental import pallas as pl

_B_TILE = 8192


def _body(a_ref, we1, be1, we2, be2, we3, be3, wd1, bd1, wd2, bd2, wd3, bd3,
          cb_ref, out_ref, idx_ref, rsum_ref, qsum_ref):
    a = a_ref[...]
    h = jax.nn.gelu(jnp.dot(a, we1[...]) + be1[...])
    h = jax.nn.gelu(jnp.dot(h, we2[...]) + be2[...])
    z = jnp.dot(h, we3[...]) + be3[...]
    zz = jnp.sum(z * z, axis=-1, keepdims=True)
    rr = 1.0 / (jnp.sqrt(zz) + 1e-8)
    z_e = z * rr
    s = zz * (rr * rr)

    cb = cb_ref[...]
    cb = cb / (jnp.sqrt(jnp.sum(cb * cb, axis=-1, keepdims=True)) + 1e-8)
    n_codes = cb.shape[0]
    half_c = 0.5 * jnp.sum(cb * cb, axis=-1)

    score = jnp.dot(z_e, cb.T) - half_c[None, :]
    m = jnp.max(score, axis=-1, keepdims=True)
    one_hot = (score == m).astype(jnp.float32)

    cbd = jax.nn.gelu(jnp.dot(cb, wd1[...]) + bd1[...])
    cbd = jax.nn.gelu(jnp.dot(cbd, wd2[...]) + bd2[...])
    cb_hat = jnp.dot(cbd, wd3[...]) + bd3[...]
    a_dim = cb_hat.shape[1]
    code_col = jax.lax.broadcasted_iota(jnp.int32, (n_codes, 1), 0).astype(
        jnp.float32)
    zpad = jnp.zeros((n_codes, 64 - a_dim), jnp.float32)
    pad = jnp.zeros((n_codes, 63), jnp.float32)
    cb_ext = jnp.concatenate([cb_hat, zpad, code_col, pad], axis=1)
    gathered = jnp.dot(one_hot, cb_ext)
    a_hat = gathered[:, :a_dim]
    idx = gathered[:, 64].astype(jnp.int32)

    out_ref[...] = a_hat
    idx_ref[...] = idx[:, None]

    r_part = jnp.reshape(jnp.sum((a - a_hat) ** 2), (1, 1))
    q_part = jnp.reshape(jnp.sum(s - 2.0 * m), (1, 1))

    @pl.when(pl.program_id(0) == 0)
    def _init():
        rsum_ref[...] = jnp.zeros((1, 1), jnp.float32)
        qsum_ref[...] = jnp.zeros((1, 1), jnp.float32)

    rsum_ref[...] += r_part
    qsum_ref[...] += q_part


def kernel(action, We1, be1, We2, be2, We3, be3, Wd1, bd1, Wd2, bd2, Wd3, bd3,
           codebook):
    n_rows, a_dim = action.shape
    n_codes, l_dim = codebook.shape
    grid = (n_rows // _B_TILE,)

    def _const2(shape):
        return pl.BlockSpec(shape, lambda i: (0, 0))

    biases = [b.reshape(1, -1) for b in (be1, be2, be3, bd1, bd2, bd3)]
    w_list = [We1, We2, We3, Wd1, Wd2, Wd3]
    in_specs = [pl.BlockSpec((_B_TILE, a_dim), lambda i: (i, 0))]
    operands = [action]
    for w, b in zip(w_list, biases):
        in_specs += [_const2(w.shape), _const2(b.shape)]
        operands += [w, b]
    in_specs.append(_const2(codebook.shape))
    operands.append(codebook)

    out_shapes = (
        jax.ShapeDtypeStruct((n_rows, a_dim), jnp.float32),
        jax.ShapeDtypeStruct((n_rows, 1), jnp.int32),
        jax.ShapeDtypeStruct((1, 1), jnp.float32),
        jax.ShapeDtypeStruct((1, 1), jnp.float32),
    )
    out_specs = (
        pl.BlockSpec((_B_TILE, a_dim), lambda i: (i, 0)),
        pl.BlockSpec((_B_TILE, 1), lambda i: (i, 0)),
        _const2((1, 1)),
        _const2((1, 1)),
    )

    a_hat, idx, rsum, qsum = pl.pallas_call(
        _body,
        grid=grid,
        in_specs=in_specs,
        out_specs=out_specs,
        out_shape=out_shapes,
    )(*operands)

    recon_loss = rsum[0, 0] / (n_rows * a_dim)
    q_loss = qsum[0, 0] / (n_rows * l_dim)
    return (a_hat, idx.reshape(n_rows), recon_loss, q_loss, q_loss)
